# initial kernel scaffold (unmeasured)
import jax
import jax.numpy as jnp
from jax import lax
from jax.experimental import pallas as pl
from jax.experimental.pallas import tpu as pltpu


def kernel(
    x,
):
    def body(*refs):
        pass

    out_shape = jax.ShapeDtypeStruct(..., jnp.float32)
    return pl.pallas_call(body, out_shape=out_shape)(...)


# baseline (device time: 71371 ns/iter reference)
import functools

import jax
import jax.numpy as jnp
from jax import lax
from jax.experimental import pallas as pl
from jax.experimental.pallas import tpu as pltpu


def kernel(x):
    x = x.reshape(512, 512)
    m, n = x.shape

    def body(x_ref, out_ref, recv_ref, send_sems, recv_sems):
        my_x = lax.axis_index("x")
        my_y = lax.axis_index("y")
        my_z = lax.axis_index("z")

        peers = [
            (1 - my_x, my_y, my_z),
            (my_x, 1 - my_y, my_z),
            (my_x, my_y, my_z ^ 1),
            (my_x, my_y, my_z ^ 2),
        ]

        barrier_sem = pltpu.get_barrier_semaphore()
        for p in peers:
            pl.semaphore_signal(
                barrier_sem, inc=1, device_id=p,
                device_id_type=pl.DeviceIdType.MESH,
            )
        pl.semaphore_wait(barrier_sem, len(peers))

        out_ref[...] = x_ref[...]

        for ph, p in enumerate(peers):
            rdma = pltpu.make_async_remote_copy(
                src_ref=out_ref,
                dst_ref=recv_ref.at[ph],
                send_sem=send_sems.at[ph],
                recv_sem=recv_sems.at[ph],
                device_id=p,
                device_id_type=pl.DeviceIdType.MESH,
            )
            rdma.start()
            rdma.wait()
            out_ref[...] = out_ref[...] + recv_ref[ph]

        @functools.partial(pl.run_scoped, sem=pltpu.SemaphoreType.REGULAR)
        def _(sem):
            for p in peers:
                pl.semaphore_signal(
                    sem, inc=1, device_id=p,
                    device_id_type=pl.DeviceIdType.MESH,
                )
            pl.semaphore_wait(sem, len(peers))

    return pl.pallas_call(
        body,
        out_shape=jax.ShapeDtypeStruct((m, n), jnp.float32),
        in_specs=[pl.BlockSpec(memory_space=pltpu.VMEM)],
        out_specs=pl.BlockSpec(memory_space=pltpu.VMEM),
        scratch_shapes=[
            pltpu.VMEM((4, m, n), jnp.float32),
            pltpu.SemaphoreType.DMA((4,)),
            pltpu.SemaphoreType.DMA((4,)),
        ],
        compiler_params=pltpu.CompilerParams(collective_id=0),
    )(x)


# device time: 33492 ns/iter; 2.1310x vs baseline; 2.1310x over previous
import functools

import jax
import jax.numpy as jnp
from jax import lax
from jax.experimental import pallas as pl
from jax.experimental.pallas import tpu as pltpu

N_CHUNK = 4
N_STEP = 4


def kernel(x):
    x = x.reshape(512, 512)
    m, n = x.shape
    rows = m // N_CHUNK

    def body(x_ref, out_ref, recv_ref, send_sems, recv_sems):
        my_x = lax.axis_index("x")
        my_y = lax.axis_index("y")
        my_z = lax.axis_index("z")

        peers = [
            (1 - my_x, my_y, my_z),
            (my_x, 1 - my_y, my_z),
            (my_x, my_y, my_z ^ 1),
            (my_x, my_y, my_z ^ 2),
        ]

        barrier_sem = pltpu.get_barrier_semaphore()
        for p in peers:
            pl.semaphore_signal(
                barrier_sem, inc=1, device_id=p,
                device_id_type=pl.DeviceIdType.MESH,
            )
        pl.semaphore_wait(barrier_sem, len(peers))

        out_ref[...] = x_ref[...]

        def make(c, s):
            return pltpu.make_async_remote_copy(
                src_ref=out_ref.at[pl.ds(c * rows, rows), :],
                dst_ref=recv_ref.at[s, c],
                send_sem=send_sems.at[s, c],
                recv_sem=recv_sems.at[s, c],
                device_id=peers[(c + s) % N_STEP],
                device_id_type=pl.DeviceIdType.MESH,
            )

        rdmas = {}
        for c in range(N_CHUNK):
            rdmas[(c, 0)] = make(c, 0)
            rdmas[(c, 0)].start()
        for s in range(N_STEP):
            for c in range(N_CHUNK):
                rdmas[(c, s)].wait()
                out_ref[pl.ds(c * rows, rows), :] = (
                    out_ref[pl.ds(c * rows, rows), :] + recv_ref[s, c]
                )
                if s + 1 < N_STEP:
                    rdmas[(c, s + 1)] = make(c, s + 1)
                    rdmas[(c, s + 1)].start()

        @functools.partial(pl.run_scoped, sem=pltpu.SemaphoreType.REGULAR)
        def _(sem):
            for p in peers:
                pl.semaphore_signal(
                    sem, inc=1, device_id=p,
                    device_id_type=pl.DeviceIdType.MESH,
                )
            pl.semaphore_wait(sem, len(peers))

    return pl.pallas_call(
        body,
        out_shape=jax.ShapeDtypeStruct((m, n), jnp.float32),
        in_specs=[pl.BlockSpec(memory_space=pltpu.VMEM)],
        out_specs=pl.BlockSpec(memory_space=pltpu.VMEM),
        scratch_shapes=[
            pltpu.VMEM((N_STEP, N_CHUNK, rows, n), jnp.float32),
            pltpu.SemaphoreType.DMA((N_STEP, N_CHUNK)),
            pltpu.SemaphoreType.DMA((N_STEP, N_CHUNK)),
        ],
        compiler_params=pltpu.CompilerParams(collective_id=0),
    )(x)


# device time: 28714 ns/iter; 2.4856x vs baseline; 1.1664x over previous
import functools

import jax
import jax.numpy as jnp
from jax import lax
from jax.experimental import pallas as pl
from jax.experimental.pallas import tpu as pltpu

N_CHUNK = 8

_Z = ("Z1", "Z2", "Z3")
ORDERS = [
    ("X", "Y") + _Z,
    ("Y", "X") + _Z,
    _Z + ("X", "Y"),
    ("Y",) + _Z + ("X",),
    _Z + ("Y", "X"),
    ("X",) + _Z + ("Y",),
    ("X", "Y") + _Z,
    _Z + ("X", "Y"),
]

SLOT = {"X": 0, "Y": 1, "Z1": 2, "Z2": 3, "Z3": 2}


def kernel(x):
    x = x.reshape(512, 512)
    m, n = x.shape
    rows = m // N_CHUNK

    def body(x_ref, out_ref, recv_ref, send_sems, recv_sems):
        my_x = lax.axis_index("x")
        my_y = lax.axis_index("y")
        my_z = lax.axis_index("z")
        is_inner = (my_z == 1) | (my_z == 2)

        x_peer = (1 - my_x, my_y, my_z)
        y_peer = (my_x, 1 - my_y, my_z)
        z_nbr = (my_x, my_y, my_z ^ 1)
        z_pair = (my_x, my_y, 3 - my_z)

        barrier_sem = pltpu.get_barrier_semaphore()
        for p in (x_peer, y_peer, z_nbr):
            pl.semaphore_signal(
                barrier_sem, inc=1, device_id=p,
                device_id_type=pl.DeviceIdType.MESH,
            )

        @pl.when(is_inner)
        def _():
            pl.semaphore_signal(
                barrier_sem, inc=1, device_id=z_pair,
                device_id_type=pl.DeviceIdType.MESH,
            )

        @pl.when(~is_inner)
        def _():
            pl.semaphore_signal(barrier_sem, inc=1)

        pl.semaphore_wait(barrier_sem, 4)

        out_ref[...] = x_ref[...]

        def chunk(c):
            return out_ref.at[pl.ds(c * rows, rows), :]

        def desc(c, tag, peer):
            s = SLOT[tag]
            return pltpu.make_async_remote_copy(
                src_ref=chunk(c),
                dst_ref=recv_ref.at[c, s],
                send_sem=send_sems.at[c, s],
                recv_sem=recv_sems.at[c, s],
                device_id=peer,
                device_id_type=pl.DeviceIdType.MESH,
            )

        def start_op(c, tag):
            if tag in ("X", "Y"):
                d = desc(c, tag, x_peer if tag == "X" else y_peer)
                d.start()
            elif tag == "Z1":
                d = desc(c, tag, z_nbr)

                @pl.when(~is_inner)
                def _():
                    d.start()

            elif tag == "Z2":
                d = desc(c, tag, z_pair)

                @pl.when(is_inner)
                def _():
                    d.start()

            else:
                d = desc(c, tag, z_nbr)

                @pl.when(is_inner)
                def _():
                    d.start()

            return d

        def complete_op(c, tag, d):
            s = SLOT[tag]
            if tag in ("X", "Y"):
                d.wait()
                chunk(c)[...] = chunk(c)[...] + recv_ref[c, s]
            elif tag == "Z1":

                @pl.when(~is_inner)
                def _():
                    d.wait_send()

                @pl.when(is_inner)
                def _():
                    d.wait_recv()
                    chunk(c)[...] = chunk(c)[...] + recv_ref[c, s]

            elif tag == "Z2":

                @pl.when(is_inner)
                def _():
                    d.wait()
                    chunk(c)[...] = chunk(c)[...] + recv_ref[c, s]

            else:

                @pl.when(is_inner)
                def _():
                    d.wait_send()

                @pl.when(~is_inner)
                def _():
                    d.wait_recv()
                    chunk(c)[...] = recv_ref[c, s]

        descs = {}
        n_ops = len(ORDERS[0])
        for k in range(n_ops):
            for c in range(N_CHUNK):
                if k > 0:
                    complete_op(c, ORDERS[c][k - 1], descs[(c, k - 1)])
                descs[(c, k)] = start_op(c, ORDERS[c][k])
        for c in range(N_CHUNK):
            complete_op(c, ORDERS[c][n_ops - 1], descs[(c, n_ops - 1)])

        @functools.partial(pl.run_scoped, sem=pltpu.SemaphoreType.REGULAR)
        def _(sem):
            for p in (x_peer, y_peer, z_nbr):
                pl.semaphore_signal(
                    sem, inc=1, device_id=p,
                    device_id_type=pl.DeviceIdType.MESH,
                )

            @pl.when(is_inner)
            def _():
                pl.semaphore_signal(
                    sem, inc=1, device_id=z_pair,
                    device_id_type=pl.DeviceIdType.MESH,
                )

            @pl.when(~is_inner)
            def _():
                pl.semaphore_signal(sem, inc=1)

            pl.semaphore_wait(sem, 4)

    return pl.pallas_call(
        body,
        out_shape=jax.ShapeDtypeStruct((m, n), jnp.float32),
        in_specs=[pl.BlockSpec(memory_space=pltpu.VMEM)],
        out_specs=pl.BlockSpec(memory_space=pltpu.VMEM),
        scratch_shapes=[
            pltpu.VMEM((N_CHUNK, 4, rows, n), jnp.float32),
            pltpu.SemaphoreType.DMA((N_CHUNK, 4)),
            pltpu.SemaphoreType.DMA((N_CHUNK, 4)),
        ],
        compiler_params=pltpu.CompilerParams(collective_id=0),
    )(x)


# device time: 27421 ns/iter; 2.6028x vs baseline; 1.0472x over previous
import functools

import jax
import jax.numpy as jnp
from jax import lax
from jax.experimental import pallas as pl
from jax.experimental.pallas import tpu as pltpu

N_CHUNK = 8

_Z = ("Z1", "Z2", "Z3")
ORDERS = [
    ("X", "Y") + _Z,
    ("Y", "X") + _Z,
    _Z + ("X", "Y"),
    ("Y",) + _Z + ("X",),
    _Z + ("Y", "X"),
    ("X",) + _Z + ("Y",),
    ("X", "Y") + _Z,
    _Z + ("X", "Y"),
]

SLOT = {"X": 0, "Y": 1, "Z1": 2, "Z2": 3, "Z3": 2}


def kernel(x):
    x = x.reshape(512, 512)
    m, n = x.shape
    rows = m // N_CHUNK

    def body(x_ref, out_ref, recv_ref, send_sems, recv_sems):
        my_x = lax.axis_index("x")
        my_y = lax.axis_index("y")
        my_z = lax.axis_index("z")
        is_inner = (my_z == 1) | (my_z == 2)

        x_peer = (1 - my_x, my_y, my_z)
        y_peer = (my_x, 1 - my_y, my_z)
        z_nbr = (my_x, my_y, my_z ^ 1)
        z_pair = (my_x, my_y, 3 - my_z)

        barrier_sem = pltpu.get_barrier_semaphore()
        for p in (x_peer, y_peer, z_nbr):
            pl.semaphore_signal(
                barrier_sem, inc=1, device_id=p,
                device_id_type=pl.DeviceIdType.MESH,
            )

        @pl.when(is_inner)
        def _():
            pl.semaphore_signal(
                barrier_sem, inc=1, device_id=z_pair,
                device_id_type=pl.DeviceIdType.MESH,
            )

        @pl.when(~is_inner)
        def _():
            pl.semaphore_signal(barrier_sem, inc=1)

        pl.semaphore_wait(barrier_sem, 4)

        def chunk(c):
            return out_ref.at[pl.ds(c * rows, rows), :]

        def in_chunk(c):
            return x_ref.at[pl.ds(c * rows, rows), :]

        def desc(c, k, peer):
            s = SLOT[ORDERS[c][k]]
            return pltpu.make_async_remote_copy(
                src_ref=in_chunk(c) if k == 0 else chunk(c),
                dst_ref=recv_ref.at[c, s],
                send_sem=send_sems.at[c, s],
                recv_sem=recv_sems.at[c, s],
                device_id=peer,
                device_id_type=pl.DeviceIdType.MESH,
            )

        def start_op(c, k):
            tag = ORDERS[c][k]
            if tag in ("X", "Y"):
                d = desc(c, k, x_peer if tag == "X" else y_peer)
                d.start()
            elif tag == "Z1":
                d = desc(c, k, z_nbr)

                @pl.when(~is_inner)
                def _():
                    d.start()

            elif tag == "Z2":
                d = desc(c, k, z_pair)

                @pl.when(is_inner)
                def _():
                    d.start()

            else:
                d = desc(c, k, z_nbr)

                @pl.when(is_inner)
                def _():
                    d.start()

            return d

        def complete_op(c, k, d):
            tag = ORDERS[c][k]
            s = SLOT[tag]
            base = in_chunk(c) if k == 0 else chunk(c)
            if tag in ("X", "Y"):
                d.wait()
                chunk(c)[...] = base[...] + recv_ref[c, s]
            elif tag == "Z1":

                @pl.when(~is_inner)
                def _():
                    d.wait_send()

                @pl.when(is_inner)
                def _():
                    d.wait_recv()
                    chunk(c)[...] = base[...] + recv_ref[c, s]

            elif tag == "Z2":

                @pl.when(is_inner)
                def _():
                    d.wait()
                    chunk(c)[...] = chunk(c)[...] + recv_ref[c, s]

            else:

                @pl.when(is_inner)
                def _():
                    d.wait_send()

                @pl.when(~is_inner)
                def _():
                    d.wait_recv()
                    chunk(c)[...] = recv_ref[c, s]

        descs = {}
        n_ops = len(ORDERS[0])
        for k in range(n_ops):
            for c in range(N_CHUNK):
                if k > 0:
                    complete_op(c, k - 1, descs[(c, k - 1)])
                descs[(c, k)] = start_op(c, k)
        for c in range(N_CHUNK):
            complete_op(c, n_ops - 1, descs[(c, n_ops - 1)])

        @functools.partial(pl.run_scoped, sem=pltpu.SemaphoreType.REGULAR)
        def _(sem):
            for p in (x_peer, y_peer, z_nbr):
                pl.semaphore_signal(
                    sem, inc=1, device_id=p,
                    device_id_type=pl.DeviceIdType.MESH,
                )

            @pl.when(is_inner)
            def _():
                pl.semaphore_signal(
                    sem, inc=1, device_id=z_pair,
                    device_id_type=pl.DeviceIdType.MESH,
                )

            @pl.when(~is_inner)
            def _():
                pl.semaphore_signal(sem, inc=1)

            pl.semaphore_wait(sem, 4)

    return pl.pallas_call(
        body,
        out_shape=jax.ShapeDtypeStruct((m, n), jnp.float32),
        in_specs=[pl.BlockSpec(memory_space=pltpu.VMEM)],
        out_specs=pl.BlockSpec(memory_space=pltpu.VMEM),
        scratch_shapes=[
            pltpu.VMEM((N_CHUNK, 4, rows, n), jnp.float32),
            pltpu.SemaphoreType.DMA((N_CHUNK, 4)),
            pltpu.SemaphoreType.DMA((N_CHUNK, 4)),
        ],
        compiler_params=pltpu.CompilerParams(collective_id=0),
    )(x)
